# baseline (speedup 1.0000x reference)
import functools

import jax
import jax.numpy as jnp
from jax import lax
from jax.experimental import pallas as pl
from jax.experimental.pallas import tpu as pltpu
from jax.experimental.pallas import tpu_sc as plsc

_TWO48 = float(2 ** 48 - 1)
_BN = 10000
_BE = 16000


def _enc_body(names_ref, w1_ref, b1_ref, w2_ref, b2_ref, out_ref, outbf_ref):
    norm = jnp.clip(names_ref[...].astype(jnp.float32) / _TWO48, 0.0, 1.0)
    h = jnp.maximum(norm * w1_ref[...] + b1_ref[...], 0.0)
    x0 = jnp.dot(h, w2_ref[...],
                 preferred_element_type=jnp.float32) + b2_ref[...]
    out_ref[...] = x0
    outbf_ref[...] = x0.astype(jnp.bfloat16)


def _encode_nodes(names, w1, b1, w2, b2):
    n = names.shape[0]
    return pl.pallas_call(
        _enc_body,
        grid=(n // _BN,),
        in_specs=[
            pl.BlockSpec((_BN, 1), lambda i: (i, 0)),
            pl.BlockSpec((1, 32), lambda i: (0, 0)),
            pl.BlockSpec((1, 32), lambda i: (0, 0)),
            pl.BlockSpec((32, 32), lambda i: (0, 0)),
            pl.BlockSpec((1, 32), lambda i: (0, 0)),
        ],
        out_specs=[
            pl.BlockSpec((_BN, 32), lambda i: (i, 0)),
            pl.BlockSpec((_BN, 32), lambda i: (i, 0)),
        ],
        out_shape=[
            jax.ShapeDtypeStruct((n, 32), jnp.float32),
            jax.ShapeDtypeStruct((n, 32), jnp.bfloat16),
        ],
    )(names.reshape(n, 1), w1, b1.reshape(1, -1), w2, b2.reshape(1, -1))


def _edge_body(a_ref, ew1_ref, eb1_ref, ew2_ref, eb2_ref,
               l1w_ref, l1b_ref, l2w_ref, l2b_ref, el1_ref, el2_ref):
    a = a_ref[...]
    h = jnp.maximum(a * ew1_ref[...] + eb1_ref[...], 0.0)
    e = jnp.dot(h, ew2_ref[...], preferred_element_type=jnp.float32) + eb2_ref[...]
    el1_ref[...] = (jnp.dot(e, l1w_ref[...], preferred_element_type=jnp.float32)
                    + l1b_ref[...]).astype(jnp.bfloat16)
    el2 = jnp.dot(e, l2w_ref[...],
                  preferred_element_type=jnp.float32) + l2b_ref[...]
    el2_ref[0, :, :] = el2[:, :32].astype(jnp.bfloat16)
    el2_ref[1, :, :] = el2[:, 32:].astype(jnp.bfloat16)


def _edge_mlps(edge_attr, ew1, eb1, ew2, eb2, l1w, l1b, l2w, l2b):
    e_cnt = edge_attr.shape[0]
    return pl.pallas_call(
        _edge_body,
        grid=(e_cnt // _BE,),
        in_specs=[
            pl.BlockSpec((_BE, 1), lambda i: (i, 0)),
            pl.BlockSpec((1, 32), lambda i: (0, 0)),
            pl.BlockSpec((1, 32), lambda i: (0, 0)),
            pl.BlockSpec((32, 32), lambda i: (0, 0)),
            pl.BlockSpec((1, 32), lambda i: (0, 0)),
            pl.BlockSpec((32, 32), lambda i: (0, 0)),
            pl.BlockSpec((1, 32), lambda i: (0, 0)),
            pl.BlockSpec((32, 64), lambda i: (0, 0)),
            pl.BlockSpec((1, 64), lambda i: (0, 0)),
        ],
        out_specs=[
            pl.BlockSpec((_BE, 32), lambda i: (i, 0)),
            pl.BlockSpec((2, _BE, 32), lambda i: (0, i, 0)),
        ],
        out_shape=[
            jax.ShapeDtypeStruct((e_cnt, 32), jnp.bfloat16),
            jax.ShapeDtypeStruct((2, e_cnt, 32), jnp.bfloat16),
        ],
    )(edge_attr, ew1, eb1.reshape(1, -1), ew2, eb2.reshape(1, -1),
      l1w, l1b.reshape(1, -1), l2w, l2b.reshape(1, -1))


def _node1_body(x_ref, aggr_ref, w1_ref, b1_ref, w2_ref, b2_ref,
                out_ref, outbf_ref):
    aggr = (aggr_ref[0, :, :].astype(jnp.float32)
            + aggr_ref[1, :, :].astype(jnp.float32))
    y = x_ref[...] + aggr
    h = jnp.maximum(jnp.dot(y, w1_ref[...],
                            preferred_element_type=jnp.float32) + b1_ref[...], 0.0)
    x1 = jnp.maximum(jnp.dot(h, w2_ref[...],
                             preferred_element_type=jnp.float32) + b2_ref[...], 0.0)
    out_ref[0, :, :] = x1[:, :32]
    out_ref[1, :, :] = x1[:, 32:]
    outbf_ref[0, :, :] = x1[:, :32].astype(jnp.bfloat16)
    outbf_ref[1, :, :] = x1[:, 32:].astype(jnp.bfloat16)


def _node_update1(x0, aggr1, w1, b1, w2, b2):
    n = x0.shape[0]
    return pl.pallas_call(
        _node1_body,
        grid=(n // _BN,),
        in_specs=[
            pl.BlockSpec((_BN, 32), lambda i: (i, 0)),
            pl.BlockSpec((2, _BN, 32), lambda i: (0, i, 0)),
            pl.BlockSpec((32, 64), lambda i: (0, 0)),
            pl.BlockSpec((1, 64), lambda i: (0, 0)),
            pl.BlockSpec((64, 64), lambda i: (0, 0)),
            pl.BlockSpec((1, 64), lambda i: (0, 0)),
        ],
        out_specs=[
            pl.BlockSpec((2, _BN, 32), lambda i: (0, i, 0)),
            pl.BlockSpec((2, _BN, 32), lambda i: (0, i, 0)),
        ],
        out_shape=[
            jax.ShapeDtypeStruct((2, n, 32), jnp.float32),
            jax.ShapeDtypeStruct((2, n, 32), jnp.bfloat16),
        ],
    )(x0, aggr1, w1, b1.reshape(1, -1), w2, b2.reshape(1, -1))


def _node2_pool_body(x1_ref, aggr_ref, batch_ref, w1_ref, b1_ref, w2_ref, b2_ref,
                     out_ref):
    y = jnp.concatenate(
        [x1_ref[0, :, :] + aggr_ref[0, :, :].astype(jnp.float32),
         x1_ref[1, :, :] + aggr_ref[1, :, :].astype(jnp.float32)], axis=1)
    h = jnp.maximum(jnp.dot(y, w1_ref[...],
                            preferred_element_type=jnp.float32) + b1_ref[...], 0.0)
    x2 = jnp.maximum(jnp.dot(h, w2_ref[...],
                             preferred_element_type=jnp.float32) + b2_ref[...], 0.0)
    aug = jnp.concatenate([x2, jnp.ones((x2.shape[0], 1), jnp.float32)], axis=1)
    ids = lax.broadcasted_iota(jnp.int32, (x2.shape[0], 64), 1)
    onehot = (batch_ref[...] == ids).astype(jnp.float32)
    contrib = lax.dot_general(onehot, aug, (((0,), (0,)), ((), ())),
                              preferred_element_type=jnp.float32)

    @pl.when(pl.program_id(0) == 0)
    def _():
        out_ref[...] = jnp.zeros_like(out_ref)

    out_ref[...] += contrib


def _node_update2_pool(x1s, aggr2, batch, w1, b1, w2, b2):
    n = batch.shape[0]
    return pl.pallas_call(
        _node2_pool_body,
        grid=(n // _BN,),
        in_specs=[
            pl.BlockSpec((2, _BN, 32), lambda i: (0, i, 0)),
            pl.BlockSpec((2, _BN, 32), lambda i: (0, i, 0)),
            pl.BlockSpec((_BN, 1), lambda i: (i, 0)),
            pl.BlockSpec((64, 64), lambda i: (0, 0)),
            pl.BlockSpec((1, 64), lambda i: (0, 0)),
            pl.BlockSpec((64, 64), lambda i: (0, 0)),
            pl.BlockSpec((1, 64), lambda i: (0, 0)),
        ],
        out_specs=pl.BlockSpec((64, 65), lambda i: (0, 0)),
        out_shape=jax.ShapeDtypeStruct((64, 65), jnp.float32),
    )(x1s, aggr2, batch.reshape(n, 1), w1, b1.reshape(1, -1), w2, b2.reshape(1, -1))


def _final_body(s_ref, g_ref, depth_ref, w1_ref, b1_ref, w2_ref, b2_ref, out_ref):
    s_mean = s_ref[:, :64] / jnp.maximum(s_ref[:, 64:65], 1.0)
    g_mean = g_ref[:, :64] / jnp.maximum(g_ref[:, 64:65], 1.0)
    z = jnp.concatenate([s_mean, g_mean, depth_ref[...]], axis=1)
    h = jnp.maximum(jnp.dot(z, w1_ref[...],
                            preferred_element_type=jnp.float32) + b1_ref[...], 0.0)
    out_ref[...] = jnp.dot(h, w2_ref[...],
                           preferred_element_type=jnp.float32) + b2_ref[...]


def _final_mlp(s_aug, g_aug, depth, w1, b1, w2, b2):
    b = depth.shape[0]
    out = pl.pallas_call(
        _final_body,
        out_shape=jax.ShapeDtypeStruct((b, 1), jnp.float32),
    )(s_aug, g_aug, depth.reshape(b, 1), w1, b1.reshape(1, -1), w2,
      b2.reshape(1, -1))
    return out.reshape(b)


_K = 1000


@functools.lru_cache(maxsize=None)
def _make_conv(etot, ntab, n, local):
    ec = etot // 2
    es = ec // 16
    nchunks = es // _K
    npad = ((n + 127) // 128) * 128
    rows_per = npad // 16
    mesh = plsc.VectorSubcoreMesh(core_axis_name="c", subcore_axis_name="s")

    def body(srck, dstk, elka, elkb, table, out,
             acc, src_v, dst_v, el_v, rows_v, sem):
        c = lax.axis_index("c")
        s = lax.axis_index("s")

        @plsc.parallel_loop(0, _K, unroll=8)
        def _(i):
            rows_v[i, :] = jnp.zeros((32,), jnp.bfloat16)
        row0 = s * rows_per
        zoff = 0
        while zoff < rows_per:
            zn = min(_K, rows_per - zoff)
            pltpu.sync_copy(rows_v.at[pl.ds(0, zn)],
                            acc.at[pl.ds(row0 + zoff, zn)])
            zoff += zn
        plsc.subcore_barrier()

        base = c * ec + s * es
        lbase = s * es if local else base

        def chunk(k, carry):
            off = pl.multiple_of(base + k * _K, 8)
            loff = pl.multiple_of(lbase + k * _K, 8)
            pltpu.sync_copy(srck.at[pl.ds(off, _K)], src_v)
            pltpu.sync_copy(dstk.at[pl.ds(loff, _K)], dst_v)

            @pl.when(c == 0)
            def _():
                pltpu.sync_copy(elka.at[pl.ds(loff, _K)], el_v)

            @pl.when(c == 1)
            def _():
                pltpu.sync_copy(elkb.at[pl.ds(loff, _K)], el_v)

            pltpu.async_copy(table.at[src_v], rows_v, sem).wait()

            @plsc.parallel_loop(0, _K, unroll=8)
            def _(i):
                rows_v[i, :] = jnp.maximum(rows_v[i, :] + el_v[i, :], 0.0)
            pltpu.sync_copy(rows_v, acc.at[dst_v], add=True)
            return carry
        lax.fori_loop(0, nchunks, chunk, 0)
        plsc.subcore_barrier()
        pltpu.sync_copy(acc.at[pl.ds(row0, rows_per)],
                        out.at[c, pl.ds(row0, rows_per)])

    return pl.kernel(
        body, mesh=mesh,
        compiler_params=pltpu.CompilerParams(use_tc_tiling_on_sc=False),
        out_type=jax.ShapeDtypeStruct((2, npad, 32), jnp.bfloat16),
        scratch_types=[
            pltpu.VMEM_SHARED((npad, 32), jnp.bfloat16),
            pltpu.VMEM((_K,), jnp.int32),
            pltpu.VMEM((_K,), jnp.int32),
            pltpu.VMEM((_K, 32), jnp.bfloat16),
            pltpu.VMEM((_K, 32), jnp.bfloat16),
            pltpu.SemaphoreType.DMA,
        ],
    )


def _encode_graph(names, edge_index, edge_attr, batch,
                  id_p, e_p, c1, c2):
    n = names.shape[0]
    src = edge_index[0]
    dst = edge_index[1]
    e_cnt = src.shape[0]
    x0, x0bf = _encode_nodes(names, *id_p)
    (l1w, l1b, s1w1, s1b1, s1w2, s1b2) = c1
    (l2w, l2b, s2w1, s2b1, s2w2, s2b2) = c2
    el1, el2s = _edge_mlps(edge_attr, *e_p, l1w, l1b, l2w, l2b)

    aggr1 = _make_conv(e_cnt, n, n, False)(src, dst, el1, el1, x0bf)
    x1s, x1sbf = _node_update1(x0, aggr1, s1w1, s1b1, s1w2, s1b2)

    srccat = jnp.concatenate([src, src + n])
    aggr2 = _make_conv(2 * e_cnt, 2 * n, n, True)(
        srccat, dst, el2s[0], el2s[1], x1sbf.reshape(2 * n, 32))
    return _node_update2_pool(x1s, aggr2, batch, s2w1, s2b1, s2w2, s2b2)


def kernel(state_node_names, state_edge_index, state_edge_attr, state_batch,
           goal_node_names, goal_edge_index, goal_edge_attr, goal_batch, depth,
           id_W1, id_b1, id_W2, id_b2, e_W1, e_b1, e_W2, e_b2,
           s1_linW, s1_linb, s1_W1, s1_b1, s1_W2, s1_b2,
           s2_linW, s2_linb, s2_W1, s2_b1, s2_W2, s2_b2,
           g1_linW, g1_linb, g1_W1, g1_b1, g1_W2, g1_b2,
           g2_linW, g2_linb, g2_W1, g2_b1, g2_W2, g2_b2,
           r_W1, r_b1, r_W2, r_b2):
    id_p = (id_W1, id_b1, id_W2, id_b2)
    e_p = (e_W1, e_b1, e_W2, e_b2)
    s_aug = _encode_graph(state_node_names, state_edge_index, state_edge_attr,
                          state_batch, id_p, e_p,
                          (s1_linW, s1_linb, s1_W1, s1_b1, s1_W2, s1_b2),
                          (s2_linW, s2_linb, s2_W1, s2_b1, s2_W2, s2_b2))
    g_aug = _encode_graph(goal_node_names, goal_edge_index, goal_edge_attr,
                          goal_batch, id_p, e_p,
                          (g1_linW, g1_linb, g1_W1, g1_b1, g1_W2, g1_b2),
                          (g2_linW, g2_linb, g2_W1, g2_b1, g2_W2, g2_b2))
    return _final_mlp(s_aug, g_aug, depth, r_W1, r_b1, r_W2, r_b2)


# speedup vs baseline: 1.1424x; 1.1424x over previous
import functools

import jax
import jax.numpy as jnp
from jax import lax
from jax.experimental import pallas as pl
from jax.experimental.pallas import tpu as pltpu
from jax.experimental.pallas import tpu_sc as plsc

_TWO48 = float(2 ** 48 - 1)
_BN = 10000
_BE = 16000


def _enc_body(names_ref, w1_ref, b1_ref, w2_ref, b2_ref, out_ref, outbf_ref):
    norm = jnp.clip(names_ref[...].astype(jnp.float32) / _TWO48, 0.0, 1.0)
    h = jnp.maximum(norm * w1_ref[...] + b1_ref[...], 0.0)
    x0 = jnp.dot(h, w2_ref[...],
                 preferred_element_type=jnp.float32) + b2_ref[...]
    out_ref[...] = x0
    outbf_ref[...] = x0.astype(jnp.bfloat16)


def _encode_nodes(names, w1, b1, w2, b2):
    n = names.shape[0]
    return pl.pallas_call(
        _enc_body,
        grid=(n // _BN,),
        in_specs=[
            pl.BlockSpec((_BN, 1), lambda i: (i, 0)),
            pl.BlockSpec((1, 32), lambda i: (0, 0)),
            pl.BlockSpec((1, 32), lambda i: (0, 0)),
            pl.BlockSpec((32, 32), lambda i: (0, 0)),
            pl.BlockSpec((1, 32), lambda i: (0, 0)),
        ],
        out_specs=[
            pl.BlockSpec((_BN, 32), lambda i: (i, 0)),
            pl.BlockSpec((_BN, 32), lambda i: (i, 0)),
        ],
        out_shape=[
            jax.ShapeDtypeStruct((n, 32), jnp.float32),
            jax.ShapeDtypeStruct((n, 32), jnp.bfloat16),
        ],
    )(names.reshape(n, 1), w1, b1.reshape(1, -1), w2, b2.reshape(1, -1))


def _edge_body(a_ref, ew1_ref, eb1_ref, ew2_ref, eb2_ref,
               l1w_ref, l1b_ref, l2w_ref, l2b_ref, el1_ref, el2_ref):
    a = a_ref[...]
    h = jnp.maximum(a * ew1_ref[...] + eb1_ref[...], 0.0)
    e = jnp.dot(h, ew2_ref[...], preferred_element_type=jnp.float32) + eb2_ref[...]
    el1_ref[...] = (jnp.dot(e, l1w_ref[...], preferred_element_type=jnp.float32)
                    + l1b_ref[...]).astype(jnp.bfloat16)
    el2 = jnp.dot(e, l2w_ref[...],
                  preferred_element_type=jnp.float32) + l2b_ref[...]
    el2_ref[0, :, :] = el2[:, :32].astype(jnp.bfloat16)
    el2_ref[1, :, :] = el2[:, 32:].astype(jnp.bfloat16)


def _edge_mlps(edge_attr, ew1, eb1, ew2, eb2, l1w, l1b, l2w, l2b):
    e_cnt = edge_attr.shape[0]
    return pl.pallas_call(
        _edge_body,
        grid=(e_cnt // _BE,),
        in_specs=[
            pl.BlockSpec((_BE, 1), lambda i: (i, 0)),
            pl.BlockSpec((1, 32), lambda i: (0, 0)),
            pl.BlockSpec((1, 32), lambda i: (0, 0)),
            pl.BlockSpec((32, 32), lambda i: (0, 0)),
            pl.BlockSpec((1, 32), lambda i: (0, 0)),
            pl.BlockSpec((32, 32), lambda i: (0, 0)),
            pl.BlockSpec((1, 32), lambda i: (0, 0)),
            pl.BlockSpec((32, 64), lambda i: (0, 0)),
            pl.BlockSpec((1, 64), lambda i: (0, 0)),
        ],
        out_specs=[
            pl.BlockSpec((_BE, 32), lambda i: (i, 0)),
            pl.BlockSpec((2, _BE, 32), lambda i: (0, i, 0)),
        ],
        out_shape=[
            jax.ShapeDtypeStruct((e_cnt, 32), jnp.bfloat16),
            jax.ShapeDtypeStruct((2, e_cnt, 32), jnp.bfloat16),
        ],
    )(edge_attr, ew1, eb1.reshape(1, -1), ew2, eb2.reshape(1, -1),
      l1w, l1b.reshape(1, -1), l2w, l2b.reshape(1, -1))


def _node1_body(x_ref, aggr_ref, w1_ref, b1_ref, w2_ref, b2_ref,
                out_ref, outbf_ref):
    aggr = (aggr_ref[0, :, :].astype(jnp.float32)
            + aggr_ref[1, :, :].astype(jnp.float32))
    y = x_ref[...] + aggr
    h = jnp.maximum(jnp.dot(y, w1_ref[...],
                            preferred_element_type=jnp.float32) + b1_ref[...], 0.0)
    x1 = jnp.maximum(jnp.dot(h, w2_ref[...],
                             preferred_element_type=jnp.float32) + b2_ref[...], 0.0)
    out_ref[0, :, :] = x1[:, :32]
    out_ref[1, :, :] = x1[:, 32:]
    outbf_ref[0, :, :] = x1[:, :32].astype(jnp.bfloat16)
    outbf_ref[1, :, :] = x1[:, 32:].astype(jnp.bfloat16)


def _node_update1(x0, aggr1, w1, b1, w2, b2):
    n = x0.shape[0]
    return pl.pallas_call(
        _node1_body,
        grid=(n // _BN,),
        in_specs=[
            pl.BlockSpec((_BN, 32), lambda i: (i, 0)),
            pl.BlockSpec((2, _BN, 32), lambda i: (0, i, 0)),
            pl.BlockSpec((32, 64), lambda i: (0, 0)),
            pl.BlockSpec((1, 64), lambda i: (0, 0)),
            pl.BlockSpec((64, 64), lambda i: (0, 0)),
            pl.BlockSpec((1, 64), lambda i: (0, 0)),
        ],
        out_specs=[
            pl.BlockSpec((2, _BN, 32), lambda i: (0, i, 0)),
            pl.BlockSpec((2, _BN, 32), lambda i: (0, i, 0)),
        ],
        out_shape=[
            jax.ShapeDtypeStruct((2, n, 32), jnp.float32),
            jax.ShapeDtypeStruct((2, n, 32), jnp.bfloat16),
        ],
    )(x0, aggr1, w1, b1.reshape(1, -1), w2, b2.reshape(1, -1))


def _node2_pool_body(x1_ref, aggr_ref, batch_ref, w1_ref, b1_ref, w2_ref, b2_ref,
                     out_ref):
    y = jnp.concatenate(
        [x1_ref[0, :, :] + aggr_ref[0, :, :].astype(jnp.float32),
         x1_ref[1, :, :] + aggr_ref[1, :, :].astype(jnp.float32)], axis=1)
    h = jnp.maximum(jnp.dot(y, w1_ref[...],
                            preferred_element_type=jnp.float32) + b1_ref[...], 0.0)
    x2 = jnp.maximum(jnp.dot(h, w2_ref[...],
                             preferred_element_type=jnp.float32) + b2_ref[...], 0.0)
    aug = jnp.concatenate([x2, jnp.ones((x2.shape[0], 1), jnp.float32)], axis=1)
    ids = lax.broadcasted_iota(jnp.int32, (x2.shape[0], 64), 1)
    onehot = (batch_ref[...] == ids).astype(jnp.float32)
    contrib = lax.dot_general(onehot, aug, (((0,), (0,)), ((), ())),
                              preferred_element_type=jnp.float32)

    @pl.when(pl.program_id(0) == 0)
    def _():
        out_ref[...] = jnp.zeros_like(out_ref)

    out_ref[...] += contrib


def _node_update2_pool(x1s, aggr2, batch, w1, b1, w2, b2):
    n = batch.shape[0]
    return pl.pallas_call(
        _node2_pool_body,
        grid=(n // _BN,),
        in_specs=[
            pl.BlockSpec((2, _BN, 32), lambda i: (0, i, 0)),
            pl.BlockSpec((2, _BN, 32), lambda i: (0, i, 0)),
            pl.BlockSpec((_BN, 1), lambda i: (i, 0)),
            pl.BlockSpec((64, 64), lambda i: (0, 0)),
            pl.BlockSpec((1, 64), lambda i: (0, 0)),
            pl.BlockSpec((64, 64), lambda i: (0, 0)),
            pl.BlockSpec((1, 64), lambda i: (0, 0)),
        ],
        out_specs=pl.BlockSpec((64, 65), lambda i: (0, 0)),
        out_shape=jax.ShapeDtypeStruct((64, 65), jnp.float32),
    )(x1s, aggr2, batch.reshape(n, 1), w1, b1.reshape(1, -1), w2, b2.reshape(1, -1))


def _final_body(s_ref, g_ref, depth_ref, w1_ref, b1_ref, w2_ref, b2_ref, out_ref):
    s_mean = s_ref[:, :64] / jnp.maximum(s_ref[:, 64:65], 1.0)
    g_mean = g_ref[:, :64] / jnp.maximum(g_ref[:, 64:65], 1.0)
    z = jnp.concatenate([s_mean, g_mean, depth_ref[...]], axis=1)
    h = jnp.maximum(jnp.dot(z, w1_ref[...],
                            preferred_element_type=jnp.float32) + b1_ref[...], 0.0)
    out_ref[...] = jnp.dot(h, w2_ref[...],
                           preferred_element_type=jnp.float32) + b2_ref[...]


def _final_mlp(s_aug, g_aug, depth, w1, b1, w2, b2):
    b = depth.shape[0]
    out = pl.pallas_call(
        _final_body,
        out_shape=jax.ShapeDtypeStruct((b, 1), jnp.float32),
    )(s_aug, g_aug, depth.reshape(b, 1), w1, b1.reshape(1, -1), w2,
      b2.reshape(1, -1))
    return out.reshape(b)


_K = 1000


@functools.lru_cache(maxsize=None)
def _make_conv(etot, ntab, n):
    ec = etot // 2
    es = ec // 16
    nchunks = es // _K
    npad = ((n + 127) // 128) * 128
    rows_per = npad // 16
    mesh = plsc.VectorSubcoreMesh(core_axis_name="c", subcore_axis_name="s")

    def body(srck, dstk, elk, table, out,
             acc, src_v, dst_v, el_v, rows_v, sem):
        c = lax.axis_index("c")
        s = lax.axis_index("s")

        @plsc.parallel_loop(0, _K, unroll=8)
        def _(i):
            rows_v[i, :] = jnp.zeros((32,), jnp.bfloat16)
        row0 = s * rows_per
        zoff = 0
        while zoff < rows_per:
            zn = min(_K, rows_per - zoff)
            pltpu.sync_copy(rows_v.at[pl.ds(0, zn)],
                            acc.at[pl.ds(row0 + zoff, zn)])
            zoff += zn
        plsc.subcore_barrier()

        base = c * ec + s * es

        def chunk(k, carry):
            off = pl.multiple_of(base + k * _K, 8)
            pltpu.sync_copy(srck.at[pl.ds(off, _K)], src_v)
            pltpu.sync_copy(dstk.at[pl.ds(off, _K)], dst_v)
            pltpu.sync_copy(elk.at[pl.ds(off, _K)], el_v)
            pltpu.async_copy(table.at[src_v], rows_v, sem).wait()

            @plsc.parallel_loop(0, _K, unroll=8)
            def _(i):
                rows_v[i, :] = jnp.maximum(rows_v[i, :] + el_v[i, :], 0.0)
            pltpu.sync_copy(rows_v, acc.at[dst_v], add=True)
            return carry
        lax.fori_loop(0, nchunks, chunk, 0)
        plsc.subcore_barrier()
        pltpu.sync_copy(acc.at[pl.ds(row0, rows_per)],
                        out.at[c, pl.ds(row0, rows_per)])

    return pl.kernel(
        body, mesh=mesh,
        compiler_params=pltpu.CompilerParams(use_tc_tiling_on_sc=False),
        out_type=jax.ShapeDtypeStruct((2, npad, 32), jnp.bfloat16),
        scratch_types=[
            pltpu.VMEM_SHARED((npad, 32), jnp.bfloat16),
            pltpu.VMEM((_K,), jnp.int32),
            pltpu.VMEM((_K,), jnp.int32),
            pltpu.VMEM((_K, 32), jnp.bfloat16),
            pltpu.VMEM((_K, 32), jnp.bfloat16),
            pltpu.SemaphoreType.DMA,
        ],
    )


def _encode_graph(names, edge_index, edge_attr, batch,
                  id_p, e_p, c1, c2):
    n = names.shape[0]
    src = edge_index[0]
    dst = edge_index[1]
    e_cnt = src.shape[0]
    x0, x0bf = _encode_nodes(names, *id_p)
    (l1w, l1b, s1w1, s1b1, s1w2, s1b2) = c1
    (l2w, l2b, s2w1, s2b1, s2w2, s2b2) = c2
    el1, el2s = _edge_mlps(edge_attr, *e_p, l1w, l1b, l2w, l2b)

    aggr1 = _make_conv(e_cnt, n, n)(src, dst, el1, x0bf)
    x1s, x1sbf = _node_update1(x0, aggr1, s1w1, s1b1, s1w2, s1b2)

    srccat = jnp.concatenate([src, src + n])
    dstcat = jnp.concatenate([dst, dst])
    aggr2 = _make_conv(2 * e_cnt, 2 * n, n)(
        srccat, dstcat, el2s.reshape(2 * e_cnt, 32), x1sbf.reshape(2 * n, 32))
    return _node_update2_pool(x1s, aggr2, batch, s2w1, s2b1, s2w2, s2b2)


def kernel(state_node_names, state_edge_index, state_edge_attr, state_batch,
           goal_node_names, goal_edge_index, goal_edge_attr, goal_batch, depth,
           id_W1, id_b1, id_W2, id_b2, e_W1, e_b1, e_W2, e_b2,
           s1_linW, s1_linb, s1_W1, s1_b1, s1_W2, s1_b2,
           s2_linW, s2_linb, s2_W1, s2_b1, s2_W2, s2_b2,
           g1_linW, g1_linb, g1_W1, g1_b1, g1_W2, g1_b2,
           g2_linW, g2_linb, g2_W1, g2_b1, g2_W2, g2_b2,
           r_W1, r_b1, r_W2, r_b2):
    id_p = (id_W1, id_b1, id_W2, id_b2)
    e_p = (e_W1, e_b1, e_W2, e_b2)
    s_aug = _encode_graph(state_node_names, state_edge_index, state_edge_attr,
                          state_batch, id_p, e_p,
                          (s1_linW, s1_linb, s1_W1, s1_b1, s1_W2, s1_b2),
                          (s2_linW, s2_linb, s2_W1, s2_b1, s2_W2, s2_b2))
    g_aug = _encode_graph(goal_node_names, goal_edge_index, goal_edge_attr,
                          goal_batch, id_p, e_p,
                          (g1_linW, g1_linb, g1_W1, g1_b1, g1_W2, g1_b2),
                          (g2_linW, g2_linb, g2_W1, g2_b1, g2_W2, g2_b2))
    return _final_mlp(s_aug, g_aug, depth, r_W1, r_b1, r_W2, r_b2)
